# Initial kernel scaffold; baseline (speedup 1.0000x reference)
import jax
import jax.numpy as jnp
from jax.experimental import pallas as pl


def kernel(feat, sup_indices, sup_values, W):
    raise NotImplementedError("write your pallas kernel here")


# speedup vs baseline: 2.7348x; 2.7348x over previous
import functools

import jax
import jax.numpy as jnp
from jax import lax
from jax.experimental import pallas as pl
from jax.experimental.pallas import tpu as pltpu
from jax.experimental.pallas import tpu_sc as plsc

NC = 2
NS = 16
NW = NC * NS
B = 128
L = 16


def _sc_spmm(feat, src3, dst3, val3, zeros, n, d, mc):
    rows_per_tile = n // NS
    mesh = plsc.VectorSubcoreMesh(core_axis_name="c", subcore_axis_name="s")

    @functools.partial(
        pl.kernel,
        out_type=jax.ShapeDtypeStruct((NC, n, d), jnp.float32),
        mesh=mesh,
        compiler_params=pltpu.CompilerParams(needs_layout_passes=False),
        scratch_types=[
            pltpu.VMEM((mc, B), jnp.int32),
            pltpu.VMEM((mc, B), jnp.int32),
            pltpu.VMEM((mc * B,), jnp.float32),
            pltpu.VMEM((B, d), jnp.float32),
            pltpu.VMEM_SHARED((n, d), jnp.float32),
            pltpu.SemaphoreType.DMA,
        ],
    )
    def sc_kernel(feat_hbm, src_hbm, dst_hbm, val_hbm, zeros_hbm, out_hbm,
                  src_v, dst_v, val_v, rows_v, acc_sh, sem):
        c = lax.axis_index("c")
        s = lax.axis_index("s")
        wid = s * NC + c
        row0 = s * rows_per_tile
        pltpu.sync_copy(zeros_hbm.at[pl.ds(row0, rows_per_tile)],
                        acc_sh.at[pl.ds(row0, rows_per_tile)])
        pltpu.sync_copy(src_hbm.at[wid], src_v)
        pltpu.sync_copy(dst_hbm.at[wid], dst_v)
        pltpu.sync_copy(val_hbm.at[wid], val_v)
        plsc.subcore_barrier()

        def chunk_body(j, carry):
            pltpu.async_copy(feat_hbm.at[src_v.at[j]], rows_v, sem).wait()
            j16 = jnp.full((L,), j * B, jnp.int32)

            def row_body(k, carry2):
                k16 = j16 + k
                v16 = plsc.load_gather(val_v, [k16])
                for jj in range(d // L):
                    sl = pl.ds(jj * L, L)
                    rows_v[k, sl] = rows_v[k, sl] * v16
                return carry2

            lax.fori_loop(0, B, row_body, 0, unroll=2)
            pltpu.sync_copy(rows_v, acc_sh.at[dst_v.at[j]], add=True)
            return carry

        lax.fori_loop(0, mc, chunk_body, 0)
        plsc.subcore_barrier()
        pltpu.sync_copy(acc_sh.at[pl.ds(row0, rows_per_tile)],
                        out_hbm.at[c, pl.ds(row0, rows_per_tile)])
        plsc.subcore_barrier()

    return sc_kernel(feat, src3, dst3, val3, zeros)


def _tc_dense(p0, p1, W, n, d, bn):
    def body(a_ref, b_ref, w_ref, o_ref):
        x = a_ref[...] + b_ref[...]
        y = jnp.dot(x, w_ref[...], preferred_element_type=jnp.float32)
        y = jnp.maximum(y, 0.0)
        nrm = jnp.sqrt(jnp.sum(y * y, axis=1, keepdims=True))
        o_ref[...] = y / jnp.maximum(nrm, 1e-12)

    return pl.pallas_call(
        body,
        grid=(n // bn,),
        in_specs=[
            pl.BlockSpec((bn, d), lambda i: (i, 0)),
            pl.BlockSpec((bn, d), lambda i: (i, 0)),
            pl.BlockSpec((d, d), lambda i: (0, 0)),
        ],
        out_specs=pl.BlockSpec((bn, d), lambda i: (i, 0)),
        out_shape=jax.ShapeDtypeStruct((n, d), jnp.float32),
    )(p0, p1, W)


def kernel(feat, sup_indices, sup_values, W, bn=1000):
    n, d = feat.shape
    e = sup_values.shape[0]
    dst = sup_indices[0]
    src = sup_indices[1]

    ew = -(-e // NW)
    mc = -(-ew // B)
    mc = -(-mc // 8) * 8
    e_pad = NW * mc * B
    pad = e_pad - e
    src_p = jnp.pad(src.astype(jnp.int32), (0, pad)).reshape(NW, mc, B)
    dst_p = jnp.pad(dst.astype(jnp.int32), (0, pad)).reshape(NW, mc, B)
    val_p = jnp.pad(sup_values, (0, pad)).reshape(NW, mc * B)

    n_pad = -(-n // (NS * 8)) * (NS * 8)
    zeros = jnp.zeros((n_pad, d), jnp.float32)

    partials = _sc_spmm(feat, src_p, dst_p, val_p, zeros, n_pad, d, mc)
    return _tc_dense(partials[0], partials[1], W, n, d, bn=bn)


# speedup vs baseline: 3.3489x; 1.2245x over previous
import functools

import jax
import jax.numpy as jnp
from jax import lax
from jax.experimental import pallas as pl
from jax.experimental.pallas import tpu as pltpu
from jax.experimental.pallas import tpu_sc as plsc

NC = 2
NS = 16
NW = NC * NS
B = 128
L = 16


def _sc_spmm(feat, src4, dst4, val4, zeros, n, d, mc):
    rows_per_tile = n // NS
    nq = mc // 4
    mesh = plsc.VectorSubcoreMesh(core_axis_name="c", subcore_axis_name="s")

    @functools.partial(
        pl.kernel,
        out_type=jax.ShapeDtypeStruct((NC, n, d), jnp.float32),
        mesh=mesh,
        compiler_params=pltpu.CompilerParams(needs_layout_passes=False),
        scratch_types=[
            pltpu.VMEM((2, 1, B), jnp.int32),
            pltpu.VMEM((2, 1, B), jnp.int32),
            pltpu.VMEM((2, 1, B), jnp.int32),
            pltpu.VMEM((2, 1, B), jnp.int32),
            pltpu.VMEM((2, 1, B), jnp.float32),
            pltpu.VMEM((2, 1, B), jnp.float32),
            pltpu.VMEM((B, d), jnp.float32),
            pltpu.VMEM((B, d), jnp.float32),
            pltpu.VMEM_SHARED((n, d), jnp.float32),
            pltpu.SemaphoreType.DMA,
            pltpu.SemaphoreType.DMA,
            pltpu.SemaphoreType.DMA,
            pltpu.SemaphoreType.DMA,
        ],
    )
    def sc_kernel(feat_hbm, src_hbm, dst_hbm, val_hbm, zeros_hbm, out_hbm,
                  srcA, srcB, dstA, dstB, valA, valB, rowsA, rowsB, acc_sh,
                  sgA, sgB, stA, stB):
        c = lax.axis_index("c")
        s = lax.axis_index("s")
        wid = s * NC + c
        row0 = s * rows_per_tile
        pltpu.sync_copy(zeros_hbm.at[pl.ds(row0, rows_per_tile)],
                        acc_sh.at[pl.ds(row0, rows_per_tile)])

        z16 = jnp.zeros((L,), jnp.int32)

        def stage(j0, sv, dv, vv, sem):
            pltpu.async_copy(src_hbm.at[wid, pl.ds(j0, 2)], sv, sem)
            pltpu.async_copy(dst_hbm.at[wid, pl.ds(j0, 2)], dv, sem)
            pltpu.async_copy(val_hbm.at[wid, pl.ds(j0, 2)], vv, sem)

        def wait_stage(j0, sv, dv, vv, sem):
            pltpu.make_async_copy(src_hbm.at[wid, pl.ds(j0, 2)], sv, sem).wait()
            pltpu.make_async_copy(dst_hbm.at[wid, pl.ds(j0, 2)], dv, sem).wait()
            pltpu.make_async_copy(val_hbm.at[wid, pl.ds(j0, 2)], vv, sem).wait()

        def gather(sv, ci, buf, sem):
            pltpu.async_copy(feat_hbm.at[sv.at[ci, 0]], buf, sem)

        def wait_gather(sv, ci, buf, sem):
            pltpu.make_async_copy(feat_hbm.at[sv.at[ci, 0]], buf, sem).wait()

        def scale(buf, vv, ci):
            ci16 = jnp.full((L,), ci, jnp.int32)

            def row_body(k, carry2):
                k16 = jnp.full((L,), k, jnp.int32)
                v16 = plsc.load_gather(vv, [ci16, z16, k16])
                for jj in range(d // L):
                    sl = pl.ds(jj * L, L)
                    buf[k, sl] = buf[k, sl] * v16
                return carry2

            lax.fori_loop(0, B, row_body, 0, unroll=2)

        def scatter(buf, dv, ci):
            pltpu.sync_copy(buf, acc_sh.at[dv.at[ci, 0]], add=True)

        stage(0, srcA, dstA, valA, stA)
        wait_stage(0, srcA, dstA, valA, stA)
        stage(2, srcB, dstB, valB, stB)
        gather(srcA, 0, rowsA, sgA)

        def quad_body(q, carry):
            c0 = 4 * q

            wait_gather(srcA, 0, rowsA, sgA)
            gather(srcA, 1, rowsB, sgB)
            scale(rowsA, valA, 0)
            scatter(rowsA, dstA, 0)

            wait_gather(srcA, 1, rowsB, sgB)
            wait_stage(c0 + 2, srcB, dstB, valB, stB)
            gather(srcB, 0, rowsA, sgA)
            scale(rowsB, valA, 1)
            scatter(rowsB, dstA, 1)

            @pl.when(q + 1 < nq)
            def _():
                stage(c0 + 4, srcA, dstA, valA, stA)

            wait_gather(srcB, 0, rowsA, sgA)
            gather(srcB, 1, rowsB, sgB)
            scale(rowsA, valB, 0)
            scatter(rowsA, dstB, 0)

            wait_gather(srcB, 1, rowsB, sgB)

            @pl.when(q + 1 < nq)
            def _():
                wait_stage(c0 + 4, srcA, dstA, valA, stA)
                gather(srcA, 0, rowsA, sgA)

            scale(rowsB, valB, 1)
            scatter(rowsB, dstB, 1)

            @pl.when(q + 1 < nq)
            def _():
                stage(c0 + 6, srcB, dstB, valB, stB)

            return carry

        lax.fori_loop(0, nq, quad_body, 0)
        plsc.subcore_barrier()
        pltpu.sync_copy(acc_sh.at[pl.ds(row0, rows_per_tile)],
                        out_hbm.at[c, pl.ds(row0, rows_per_tile)])
        plsc.subcore_barrier()

    return sc_kernel(feat, src4, dst4, val4, zeros)


def _tc_dense(p0, p1, W, n, d, bn):
    def body(a_ref, b_ref, w_ref, o_ref):
        x = a_ref[...] + b_ref[...]
        y = jnp.dot(x, w_ref[...], preferred_element_type=jnp.float32)
        y = jnp.maximum(y, 0.0)
        nrm = jnp.sqrt(jnp.sum(y * y, axis=1, keepdims=True))
        o_ref[...] = y / jnp.maximum(nrm, 1e-12)

    return pl.pallas_call(
        body,
        grid=(n // bn,),
        in_specs=[
            pl.BlockSpec((bn, d), lambda i: (i, 0)),
            pl.BlockSpec((bn, d), lambda i: (i, 0)),
            pl.BlockSpec((d, d), lambda i: (0, 0)),
        ],
        out_specs=pl.BlockSpec((bn, d), lambda i: (i, 0)),
        out_shape=jax.ShapeDtypeStruct((n, d), jnp.float32),
    )(p0, p1, W)


def kernel(feat, sup_indices, sup_values, W, bn=1000):
    n, d = feat.shape
    e = sup_values.shape[0]
    dst = sup_indices[0]
    src = sup_indices[1]

    ew = -(-e // NW)
    mc = -(-ew // B)
    mc = -(-mc // 8) * 8
    e_pad = NW * mc * B
    pad = e_pad - e
    src_p = jnp.pad(src.astype(jnp.int32), (0, pad)).reshape(NW, mc, 1, B)
    dst_p = jnp.pad(dst.astype(jnp.int32), (0, pad)).reshape(NW, mc, 1, B)
    val_p = jnp.pad(sup_values, (0, pad)).reshape(NW, mc, 1, B)

    n_pad = -(-n // (NS * 8)) * (NS * 8)
    zeros = jnp.zeros((n_pad, d), jnp.float32)

    partials = _sc_spmm(feat, src_p, dst_p, val_p, zeros, n_pad, d, mc)
    return _tc_dense(partials[0], partials[1], W, n, d, bn=bn)


# speedup vs baseline: 3.4768x; 1.0382x over previous
import functools

import jax
import jax.numpy as jnp
from jax import lax
from jax.experimental import pallas as pl
from jax.experimental.pallas import tpu as pltpu
from jax.experimental.pallas import tpu_sc as plsc

NC = 2
NS = 16
NW = NC * NS
B = 64
L = 16
P = 4


def _sc_spmm(feat, src4, dst4, val4, zeros, n, d, mc):
    rows_per_tile = n // NS
    n8 = mc // (2 * P)
    mesh = plsc.VectorSubcoreMesh(core_axis_name="c", subcore_axis_name="s")

    @functools.partial(
        pl.kernel,
        out_type=jax.ShapeDtypeStruct((NC, n, d), jnp.float32),
        mesh=mesh,
        compiler_params=pltpu.CompilerParams(needs_layout_passes=False),
        scratch_types=[
            pltpu.VMEM((P, 1, B), jnp.int32),
            pltpu.VMEM((P, 1, B), jnp.int32),
            pltpu.VMEM((P, 1, B), jnp.int32),
            pltpu.VMEM((P, 1, B), jnp.int32),
            pltpu.VMEM((P, 1, B), jnp.float32),
            pltpu.VMEM((P, 1, B), jnp.float32),
            pltpu.VMEM((B, d), jnp.float32),
            pltpu.VMEM((B, d), jnp.float32),
            pltpu.VMEM((B, d), jnp.float32),
            pltpu.VMEM((B, d), jnp.float32),
            pltpu.VMEM_SHARED((n, d), jnp.float32),
            pltpu.SemaphoreType.DMA,
            pltpu.SemaphoreType.DMA,
            pltpu.SemaphoreType.DMA,
            pltpu.SemaphoreType.DMA,
            pltpu.SemaphoreType.DMA,
            pltpu.SemaphoreType.DMA,
            pltpu.SemaphoreType.DMA,
            pltpu.SemaphoreType.DMA,
            pltpu.SemaphoreType.DMA,
            pltpu.SemaphoreType.DMA,
            pltpu.SemaphoreType.DMA,
        ],
    )
    def sc_kernel(feat_hbm, src_hbm, dst_hbm, val_hbm, zeros_hbm, out_hbm,
                  srcA, srcB, dstA, dstB, valA, valB,
                  rows0, rows1, rows2, rows3, acc_sh,
                  stA, stB, zsem, sg0, sg1, sg2, sg3, sc0, sc1, sc2, sc3):
        c = lax.axis_index("c")
        s = lax.axis_index("s")
        wid = s * NC + c
        row0 = s * rows_per_tile
        rows = [rows0, rows1, rows2, rows3]
        sg = [sg0, sg1, sg2, sg3]
        sc = [sc0, sc1, sc2, sc3]
        win = [(srcA, dstA, valA, stA), (srcB, dstB, valB, stB)]

        pltpu.async_copy(zeros_hbm.at[pl.ds(row0, rows_per_tile)],
                         acc_sh.at[pl.ds(row0, rows_per_tile)], zsem)

        def stage(j0, w):
            sv, dv, vv, sem = w
            pltpu.async_copy(src_hbm.at[wid, pl.ds(j0, P)], sv, sem)
            pltpu.async_copy(dst_hbm.at[wid, pl.ds(j0, P)], dv, sem)
            pltpu.async_copy(val_hbm.at[wid, pl.ds(j0, P)], vv, sem)

        def wait_stage(w):
            sv, dv, vv, sem = w
            pltpu.make_async_copy(src_hbm.at[wid, pl.ds(0, P)], sv, sem).wait()
            pltpu.make_async_copy(dst_hbm.at[wid, pl.ds(0, P)], dv, sem).wait()
            pltpu.make_async_copy(val_hbm.at[wid, pl.ds(0, P)], vv, sem).wait()

        def gather(w, wk, buf, sem):
            pltpu.async_copy(feat_hbm.at[w[0].at[wk, 0]], buf, sem)

        def wait_gather(w, wk, buf, sem):
            pltpu.make_async_copy(feat_hbm.at[w[0].at[wk, 0]], buf, sem).wait()

        def scatter(buf, w, wk, sem):
            pltpu.async_copy(buf, acc_sh.at[w[1].at[wk, 0]], sem, add=True)

        def wait_scatter(buf, w, wk, sem):
            pltpu.make_async_copy(buf, acc_sh.at[w[1].at[wk, 0]], sem).wait()

        z16 = jnp.zeros((L,), jnp.int32)

        def scale(buf, w, wk):
            wk16 = jnp.full((L,), wk, jnp.int32)
            vv = w[2]

            def row_body(k, carry2):
                k16 = jnp.full((L,), k, jnp.int32)
                v16 = plsc.load_gather(vv, [wk16, z16, k16])
                for jj in range(d // L):
                    sl = pl.ds(jj * L, L)
                    buf[k, sl] = buf[k, sl] * v16
                return carry2

            lax.fori_loop(0, B, row_body, 0, unroll=4)

        stage(0, win[0])
        wait_stage(win[0])
        pltpu.make_async_copy(zeros_hbm.at[pl.ds(row0, rows_per_tile)],
                              acc_sh.at[pl.ds(row0, rows_per_tile)],
                              zsem).wait()
        plsc.subcore_barrier()

        gather(win[0], 0, rows[0], sg[0])
        gather(win[0], 1, rows[1], sg[1])

        def oct_body(p, carry):
            c0 = 2 * P * p
            for k in range(2 * P):
                ci = c0 + k
                bk = k % P
                kn = (k + 2) % P
                cur = win[0] if k < P else win[1]
                nxt = win[(k // P + 1) % 2]

                wait_gather(cur, k % P, rows[bk], sg[bk])

                @pl.when(ci >= 2)
                def _():
                    wait_scatter(rows[kn], cur, 0, sc[kn])

                if k == 1:
                    stage(c0 + P, win[1])
                if k == 5:
                    @pl.when(c0 + 2 * P < mc)
                    def _():
                        stage(c0 + 2 * P, win[0])
                if k == 2:
                    wait_stage(win[1])
                if k == 6:
                    @pl.when(c0 + 2 * P < mc)
                    def _():
                        wait_stage(win[0])

                @pl.when(ci + 2 < mc)
                def _():
                    gather(nxt if k % P >= 2 else cur, (k + 2) % P,
                           rows[kn], sg[kn])

                scale(rows[bk], cur, k % P)
                scatter(rows[bk], cur, k % P, sc[bk])
            return carry

        lax.fori_loop(0, n8, oct_body, 0)
        wait_scatter(rows[(mc - 2) % P], win[1], 0, sc[(mc - 2) % P])
        wait_scatter(rows[(mc - 1) % P], win[1], 0, sc[(mc - 1) % P])
        plsc.subcore_barrier()
        pltpu.sync_copy(acc_sh.at[pl.ds(row0, rows_per_tile)],
                        out_hbm.at[c, pl.ds(row0, rows_per_tile)])
        plsc.subcore_barrier()

    return sc_kernel(feat, src4, dst4, val4, zeros)


def _tc_dense(p0, p1, W, n, d, bn):
    def body(a_ref, b_ref, w_ref, o_ref):
        x = a_ref[...] + b_ref[...]
        y = jnp.dot(x, w_ref[...], preferred_element_type=jnp.float32)
        y = jnp.maximum(y, 0.0)
        nrm = jnp.sqrt(jnp.sum(y * y, axis=1, keepdims=True))
        o_ref[...] = y / jnp.maximum(nrm, 1e-12)

    return pl.pallas_call(
        body,
        grid=(n // bn,),
        in_specs=[
            pl.BlockSpec((bn, d), lambda i: (i, 0)),
            pl.BlockSpec((bn, d), lambda i: (i, 0)),
            pl.BlockSpec((d, d), lambda i: (0, 0)),
        ],
        out_specs=pl.BlockSpec((bn, d), lambda i: (i, 0)),
        out_shape=jax.ShapeDtypeStruct((n, d), jnp.float32),
    )(p0, p1, W)


def kernel(feat, sup_indices, sup_values, W, bn=1000):
    n, d = feat.shape
    e = sup_values.shape[0]
    dst = sup_indices[0]
    src = sup_indices[1]

    ew = -(-e // NW)
    mc = -(-ew // B)
    mc = -(-mc // 8) * 8
    e_pad = NW * mc * B
    pad = e_pad - e
    src_p = jnp.pad(src.astype(jnp.int32), (0, pad)).reshape(NW, mc, 1, B)
    dst_p = jnp.pad(dst.astype(jnp.int32), (0, pad)).reshape(NW, mc, 1, B)
    val_p = jnp.pad(sup_values, (0, pad)).reshape(NW, mc, 1, B)

    n_pad = -(-n // (NS * 8)) * (NS * 8)
    zeros = jnp.zeros((n_pad, d), jnp.float32)

    partials = _sc_spmm(feat, src_p, dst_p, val_p, zeros, n_pad, d, mc)
    return _tc_dense(partials[0], partials[1], W, n, d, bn=bn)
